# baseline (speedup 1.0000x reference)
import functools

import jax
import jax.numpy as jnp
from jax import lax
from jax.experimental import pallas as pl
from jax.experimental.pallas import tpu as pltpu
from jax.experimental.pallas import tpu_sc as plsc

_HASH_SIZE = 1048576
_DIMENSIONS = 64
_N_QUERIES = 16384
_PRIME1 = 73856093
_PRIME2 = 19349663

_NC = 2
_NS = 16
_NW = _NC * _NS
_BPW = _N_QUERIES // _NW
_CH = 128
_NCHUNK = _BPW // _CH
_L = 16


@functools.partial(
    pl.kernel,
    out_type=jax.ShapeDtypeStruct((_N_QUERIES, _DIMENSIONS), jnp.float32),
    mesh=plsc.VectorSubcoreMesh(core_axis_name="c", subcore_axis_name="s"),
    compiler_params=pltpu.CompilerParams(use_tc_tiling_on_sc=False),
    scratch_types=[
        pltpu.VMEM((_BPW,), jnp.float32),
        pltpu.VMEM((_BPW,), jnp.float32),
        pltpu.VMEM((_NCHUNK, _CH), jnp.int32),
        pltpu.VMEM((_BPW, _DIMENSIONS), jnp.float32),
        pltpu.SemaphoreType.DMA,
    ],
)
def _hash_gather(x_hbm, y_hbm, grid_hbm, out_hbm, x_v, y_v, idx_v, rows_v, sem):
    wid = lax.axis_index("s") * _NC + lax.axis_index("c")
    base = wid * _BPW

    pltpu.sync_copy(x_hbm.at[pl.ds(base, _BPW)], x_v)
    pltpu.sync_copy(y_hbm.at[pl.ds(base, _BPW)], y_v)

    for j in range(_NCHUNK):
        for k in range(_CH // _L):
            off = j * _CH + k * _L
            ix = x_v[pl.ds(off, _L)].astype(jnp.int32)
            iy = y_v[pl.ds(off, _L)].astype(jnp.int32)
            h = (ix * _PRIME1) ^ (iy * _PRIME2)
            idx_v[j, pl.ds(k * _L, _L)] = h & (_HASH_SIZE - 1)

    copies = []
    for j in range(_NCHUNK):
        c = pltpu.make_async_copy(
            grid_hbm.at[idx_v.at[j]], rows_v.at[pl.ds(j * _CH, _CH)], sem
        )
        c.start()
        copies.append(c)
    for c in copies:
        c.wait()

    pltpu.sync_copy(rows_v, out_hbm.at[pl.ds(base, _BPW)])


def kernel(positions, grid):
    return _hash_gather(positions[:, 0], positions[:, 1], grid)


# speedup vs baseline: 9.0087x; 9.0087x over previous
import functools

import jax
import jax.numpy as jnp
from jax import lax
from jax.experimental import pallas as pl
from jax.experimental.pallas import tpu as pltpu
from jax.experimental.pallas import tpu_sc as plsc

_HASH_SIZE = 1048576
_DIMENSIONS = 64
_N_QUERIES = 16384
_PRIME1 = 73856093
_PRIME2 = 19349663

_NC = 2
_NS = 16
_NW = _NC * _NS
_L = 16
_CB = 128
_NCB = _N_QUERIES // _CB
_CPW = _NCB // _NW
_BPW = _CPW * _CB
_TDIM = _DIMENSIONS // 8
_FLAT = _HASH_SIZE * _DIMENSIONS


@functools.partial(
    pl.kernel,
    out_type=jax.ShapeDtypeStruct((_TDIM, _NCB, 8, _CB), jnp.float32),
    mesh=plsc.VectorSubcoreMesh(core_axis_name="c", subcore_axis_name="s"),
    compiler_params=pltpu.CompilerParams(use_tc_tiling_on_sc=False),
    scratch_types=[
        pltpu.VMEM((_BPW,), jnp.float32),
        pltpu.VMEM((_BPW,), jnp.float32),
        pltpu.VMEM((_CPW, _DIMENSIONS, _CB), jnp.int32),
        pltpu.VMEM((_TDIM, _CPW, 8, _CB), jnp.float32),
        pltpu.SemaphoreType.DMA,
    ],
)
def _hash_gather(x_hbm, y_hbm, gflat_hbm, out_hbm, x_v, y_v, idx_v, rows_v, sem):
    wid = lax.axis_index("s") * _NC + lax.axis_index("c")
    base = wid * _BPW

    pltpu.sync_copy(x_hbm.at[pl.ds(base, _BPW)], x_v)
    pltpu.sync_copy(y_hbm.at[pl.ds(base, _BPW)], y_v)

    copies = []
    for cb in range(_CPW):
        hterms = []
        for k in range(_CB // _L):
            off = cb * _CB + k * _L
            ix = x_v[pl.ds(off, _L)].astype(jnp.int32)
            iy = y_v[pl.ds(off, _L)].astype(jnp.int32)
            h = ((ix * _PRIME1) ^ (iy * _PRIME2)) & (_HASH_SIZE - 1)
            hterms.append((h >> 7) * 1024 + (h & 127))
        for t in range(_TDIM):
            for s in range(8):
                d = t * 8 + s
                const = t * (_HASH_SIZE * 8) + s * _CB
                for k in range(_CB // _L):
                    idx_v[cb, d, pl.ds(k * _L, _L)] = hterms[k] + const
                c = pltpu.make_async_copy(
                    gflat_hbm.at[idx_v.at[cb, d]], rows_v.at[t, cb, s], sem
                )
                c.start()
                copies.append(c)
    for c in copies:
        c.wait()

    for t in range(_TDIM):
        for cb in range(_CPW):
            pltpu.sync_copy(rows_v.at[t, cb], out_hbm.at[t, wid * _CPW + cb])


def kernel(positions, grid):
    gflat = grid.T.reshape(_TDIM, 8, _HASH_SIZE // _CB, _CB)
    gflat = gflat.transpose(0, 2, 1, 3).reshape(_FLAT)
    out4 = _hash_gather(positions[:, 0], positions[:, 1], gflat)
    return out4.transpose(0, 2, 1, 3).reshape(_DIMENSIONS, _N_QUERIES).T


# speedup vs baseline: 9.1486x; 1.0155x over previous
import functools

import jax
import jax.numpy as jnp
from jax import lax
from jax.experimental import pallas as pl
from jax.experimental.pallas import tpu as pltpu
from jax.experimental.pallas import tpu_sc as plsc

_HASH_SIZE = 1048576
_DIMENSIONS = 64
_N_QUERIES = 16384
_PRIME1 = 73856093
_PRIME2 = 19349663

_NC = 2
_NS = 16
_NW = _NC * _NS
_L = 16
_CB = 128
_NCB = _N_QUERIES // _CB
_CPW = _NCB // _NW
_BPW = _CPW * _CB
_TDIM = _DIMENSIONS // 8
_FLAT = _HASH_SIZE * _DIMENSIONS


@functools.partial(
    pl.kernel,
    out_type=jax.ShapeDtypeStruct((_TDIM, _NCB, 8, _CB), jnp.float32),
    mesh=plsc.VectorSubcoreMesh(core_axis_name="c", subcore_axis_name="s"),
    compiler_params=pltpu.CompilerParams(use_tc_tiling_on_sc=False),
    scratch_types=[
        pltpu.VMEM((_BPW,), jnp.float32),
        pltpu.VMEM((_BPW,), jnp.float32),
        pltpu.VMEM((_CPW, _DIMENSIONS, _CB), jnp.int32),
        pltpu.VMEM((_TDIM, _CPW, 8, _CB), jnp.float32),
        [pltpu.SemaphoreType.DMA] * _CPW,
        pltpu.SemaphoreType.DMA,
    ],
)
def _hash_gather(
    x_hbm, y_hbm, gflat_hbm, out_hbm, x_v, y_v, idx_v, rows_v, gsems, wsem
):
    wid = lax.axis_index("s") * _NC + lax.axis_index("c")
    base = wid * _BPW

    pltpu.sync_copy(x_hbm.at[pl.ds(base, _BPW)], x_v)
    pltpu.sync_copy(y_hbm.at[pl.ds(base, _BPW)], y_v)

    gathers = [[] for _ in range(_CPW)]
    for cb in range(_CPW):
        hterms = []
        for k in range(_CB // _L):
            off = cb * _CB + k * _L
            ix = x_v[pl.ds(off, _L)].astype(jnp.int32)
            iy = y_v[pl.ds(off, _L)].astype(jnp.int32)
            h = ((ix * _PRIME1) ^ (iy * _PRIME2)) & (_HASH_SIZE - 1)
            hterms.append((h >> 7) * 1024 + (h & 127))
        for t in range(_TDIM):
            for s in range(8):
                d = t * 8 + s
                const = t * (_HASH_SIZE * 8) + s * _CB
                for k in range(_CB // _L):
                    idx_v[cb, d, pl.ds(k * _L, _L)] = hterms[k] + const
                c = pltpu.make_async_copy(
                    gflat_hbm.at[idx_v.at[cb, d]], rows_v.at[t, cb, s], gsems[cb]
                )
                c.start()
                gathers[cb].append(c)

    writebacks = []
    for cb in range(_CPW):
        for c in gathers[cb]:
            c.wait()
        for t in range(_TDIM):
            w = pltpu.make_async_copy(
                rows_v.at[t, cb], out_hbm.at[t, wid * _CPW + cb], wsem
            )
            w.start()
            writebacks.append(w)
    for w in writebacks:
        w.wait()


def kernel(positions, grid):
    gflat = grid.T.reshape(_TDIM, 8, _HASH_SIZE // _CB, _CB)
    gflat = gflat.transpose(0, 2, 1, 3).reshape(_FLAT)
    out4 = _hash_gather(positions[:, 0], positions[:, 1], gflat)
    return out4.transpose(0, 2, 1, 3).reshape(_DIMENSIONS, _N_QUERIES).T


# speedup vs baseline: 9.2448x; 1.0105x over previous
import functools

import jax
import jax.numpy as jnp
from jax import lax
from jax.experimental import pallas as pl
from jax.experimental.pallas import tpu as pltpu
from jax.experimental.pallas import tpu_sc as plsc

_HASH_SIZE = 1048576
_DIMENSIONS = 64
_N_QUERIES = 16384
_PRIME1 = 73856093
_PRIME2 = 19349663

_NC = 2
_NS = 16
_NW = _NC * _NS
_L = 16
_CB = 128
_NCB = _N_QUERIES // _CB
_CPW = _NCB // _NW
_BPW = _CPW * _CB
_TDIM = _DIMENSIONS // 8
_FLAT = _HASH_SIZE * _DIMENSIONS


@functools.partial(
    pl.kernel,
    out_type=jax.ShapeDtypeStruct((_TDIM, _NCB, 8, _CB), jnp.float32),
    mesh=plsc.VectorSubcoreMesh(core_axis_name="c", subcore_axis_name="s"),
    compiler_params=pltpu.CompilerParams(use_tc_tiling_on_sc=False),
    scratch_types=[
        pltpu.VMEM((_CPW, 2, _CB), jnp.float32),
        pltpu.VMEM((_CPW, _DIMENSIONS, _CB), jnp.int32),
        pltpu.VMEM((_TDIM, _CPW, 8, _CB), jnp.float32),
        [pltpu.SemaphoreType.DMA] * _CPW,
        pltpu.SemaphoreType.DMA,
    ],
)
def _hash_gather(p3_hbm, gflat_hbm, out_hbm, pos_v, idx_v, rows_v, gsems, wsem):
    wid = lax.axis_index("s") * _NC + lax.axis_index("c")

    pltpu.sync_copy(p3_hbm.at[pl.ds(wid * _CPW, _CPW)], pos_v)

    gathers = [[] for _ in range(_CPW)]
    for cb in range(_CPW):
        hterms = []
        for k in range(_CB // _L):
            off = k * _L
            ix = pos_v[cb, 0, pl.ds(off, _L)].astype(jnp.int32)
            iy = pos_v[cb, 1, pl.ds(off, _L)].astype(jnp.int32)
            h = ((ix * _PRIME1) ^ (iy * _PRIME2)) & (_HASH_SIZE - 1)
            hterms.append((h >> 7) * 1024 + (h & 127))
        for t in range(_TDIM):
            for s in range(8):
                d = t * 8 + s
                const = t * (_HASH_SIZE * 8) + s * _CB
                for k in range(_CB // _L):
                    idx_v[cb, d, pl.ds(k * _L, _L)] = hterms[k] + const
                c = pltpu.make_async_copy(
                    gflat_hbm.at[idx_v.at[cb, d]], rows_v.at[t, cb, s], gsems[cb]
                )
                c.start()
                gathers[cb].append(c)

    writebacks = []
    for cb in range(_CPW):
        for c in gathers[cb]:
            c.wait()
        for t in range(_TDIM):
            w = pltpu.make_async_copy(
                rows_v.at[t, cb], out_hbm.at[t, wid * _CPW + cb], wsem
            )
            w.start()
            writebacks.append(w)
    for w in writebacks:
        w.wait()


def kernel(positions, grid):
    gflat = grid.T.reshape(_TDIM, 8, _HASH_SIZE // _CB, _CB)
    gflat = gflat.transpose(0, 2, 1, 3).reshape(_FLAT)
    p3 = positions.T.reshape(2, _NCB, _CB).transpose(1, 0, 2)
    out4 = _hash_gather(p3, gflat)
    return out4.transpose(0, 2, 1, 3).reshape(_DIMENSIONS, _N_QUERIES).T


# speedup vs baseline: 9.8545x; 1.0659x over previous
import functools

import jax
import jax.numpy as jnp
from jax import lax
from jax.experimental import pallas as pl
from jax.experimental.pallas import tpu as pltpu
from jax.experimental.pallas import tpu_sc as plsc

_HASH_SIZE = 1048576
_DIMENSIONS = 64
_N_QUERIES = 16384
_PRIME1 = 73856093
_PRIME2 = 19349663

_NC = 2
_NS = 16
_NW = _NC * _NS
_L = 16
_CB = 128
_NCB = _N_QUERIES // _CB
_CPW = _NCB // _NW
_BPW = _CPW * _CB
_TDIM = _DIMENSIONS // 8
_FLAT = _HASH_SIZE * _DIMENSIONS


@functools.partial(
    pl.kernel,
    out_type=jax.ShapeDtypeStruct((_TDIM, _NCB, 8, _CB), jnp.float32),
    mesh=plsc.VectorSubcoreMesh(core_axis_name="c", subcore_axis_name="s"),
    compiler_params=pltpu.CompilerParams(use_tc_tiling_on_sc=False),
    scratch_types=[
        pltpu.VMEM((_CPW, 2, _CB), jnp.float32),
        pltpu.VMEM((_CPW, _CB), jnp.int32),
        pltpu.VMEM((_TDIM, _CPW, 8, _CB), jnp.float32),
        [pltpu.SemaphoreType.DMA] * _CPW,
        pltpu.SemaphoreType.DMA,
    ],
)
def _hash_gather(p3_hbm, gflat_hbm, out_hbm, pos_v, idx_v, rows_v, gsems, wsem):
    wid = lax.axis_index("s") * _NC + lax.axis_index("c")

    pltpu.sync_copy(p3_hbm.at[pl.ds(wid * _CPW, _CPW)], pos_v)

    gathers = [[] for _ in range(_CPW)]
    for cb in range(_CPW):
        for k in range(_CB // _L):
            off = k * _L
            ix = pos_v[cb, 0, pl.ds(off, _L)].astype(jnp.int32)
            iy = pos_v[cb, 1, pl.ds(off, _L)].astype(jnp.int32)
            h = ((ix * _PRIME1) ^ (iy * _PRIME2)) & (_HASH_SIZE - 1)
            idx_v[cb, pl.ds(off, _L)] = (h >> 7) * 1024 + (h & 127)
        for t in range(_TDIM):
            for s in range(8):
                const = t * (_HASH_SIZE * 8) + s * _CB
                src = gflat_hbm.at[pl.ds(const, _FLAT - const)].at[idx_v.at[cb]]
                c = pltpu.make_async_copy(src, rows_v.at[t, cb, s], gsems[cb])
                c.start()
                gathers[cb].append(c)

    writebacks = []
    for cb in range(_CPW):
        for c in gathers[cb]:
            c.wait()
        for t in range(_TDIM):
            w = pltpu.make_async_copy(
                rows_v.at[t, cb], out_hbm.at[t, wid * _CPW + cb], wsem
            )
            w.start()
            writebacks.append(w)
    for w in writebacks:
        w.wait()


def kernel(positions, grid):
    gflat = grid.T.reshape(_TDIM, 8, _HASH_SIZE // _CB, _CB)
    gflat = gflat.transpose(0, 2, 1, 3).reshape(_FLAT)
    p3 = positions.T.reshape(2, _NCB, _CB).transpose(1, 0, 2)
    out4 = _hash_gather(p3, gflat)
    return out4.transpose(0, 2, 1, 3).reshape(_DIMENSIONS, _N_QUERIES).T


# speedup vs baseline: 9.8657x; 1.0011x over previous
import functools

import jax
import jax.numpy as jnp
from jax import lax
from jax.experimental import pallas as pl
from jax.experimental.pallas import tpu as pltpu
from jax.experimental.pallas import tpu_sc as plsc

_HASH_SIZE = 1048576
_DIMENSIONS = 64
_N_QUERIES = 16384
_PRIME1 = 73856093
_PRIME2 = 19349663

_NC = 2
_NS = 16
_NW = _NC * _NS
_L = 16
_CB = 128
_NCB = _N_QUERIES // _CB
_CPW = _NCB // _NW
_BPW = _CPW * _CB
_TDIM = _DIMENSIONS // 8
_FLAT = _HASH_SIZE * _DIMENSIONS


@functools.partial(
    pl.kernel,
    out_type=jax.ShapeDtypeStruct((_TDIM, _NCB, 8, _CB), jnp.float32),
    mesh=plsc.VectorSubcoreMesh(core_axis_name="c", subcore_axis_name="s"),
    compiler_params=pltpu.CompilerParams(use_tc_tiling_on_sc=False),
    scratch_types=[
        pltpu.VMEM((_CPW, 2, _CB), jnp.float32),
        pltpu.VMEM((_CPW, _CB), jnp.int32),
        pltpu.VMEM((_TDIM, _CPW, 8, _CB), jnp.float32),
        [pltpu.SemaphoreType.DMA] * _CPW,
        pltpu.SemaphoreType.DMA,
    ],
)
def _hash_gather(p3_hbm, gflat_hbm, out_hbm, pos_v, idx_v, rows_v, gsems, wsem):
    wid = lax.axis_index("s") * _NC + lax.axis_index("c")

    pltpu.sync_copy(p3_hbm.at[pl.ds(wid * _CPW, _CPW)], pos_v)

    gathers = [[] for _ in range(_CPW)]
    for cb in range(_CPW):
        for k in range(_CB // _L):
            off = k * _L
            ix = pos_v[cb, 0, pl.ds(off, _L)].astype(jnp.int32)
            iy = pos_v[cb, 1, pl.ds(off, _L)].astype(jnp.int32)
            h = ((ix * _PRIME1) ^ (iy * _PRIME2)) & (_HASH_SIZE - 1)
            idx_v[cb, pl.ds(off, _L)] = (h >> 7) * 1024 + (h & 127)
        for s in range(8):
            for t in range(_TDIM):
                const = t * (_HASH_SIZE * 8) + s * _CB
                src = gflat_hbm.at[pl.ds(const, _FLAT - const)].at[idx_v.at[cb]]
                c = pltpu.make_async_copy(src, rows_v.at[t, cb, s], gsems[cb])
                c.start()
                gathers[cb].append(c)

    writebacks = []
    for cb in range(_CPW):
        for c in gathers[cb]:
            c.wait()
        for t in range(_TDIM):
            w = pltpu.make_async_copy(
                rows_v.at[t, cb], out_hbm.at[t, wid * _CPW + cb], wsem
            )
            w.start()
            writebacks.append(w)
    for w in writebacks:
        w.wait()


def kernel(positions, grid):
    gflat = grid.T.reshape(_TDIM, 8, _HASH_SIZE // _CB, _CB)
    gflat = gflat.transpose(0, 2, 1, 3).reshape(_FLAT)
    p3 = positions.T.reshape(2, _NCB, _CB).transpose(1, 0, 2)
    out4 = _hash_gather(p3, gflat)
    return out4.transpose(0, 2, 1, 3).reshape(_DIMENSIONS, _N_QUERIES).T
